# Initial kernel scaffold; baseline (speedup 1.0000x reference)
import jax
import jax.numpy as jnp
from jax.experimental import pallas as pl


def kernel(xyz, features, params):
    raise NotImplementedError("write your pallas kernel here")


# speedup vs baseline: 1.0002x; 1.0002x over previous
import functools
import jax
import jax.numpy as jnp
import numpy as np
from jax.experimental import pallas as pl
from jax.experimental.pallas import tpu as pltpu

B, N, C = 4, 16384, 64
NPOINT = 1024
RADII = [0.5, 1.0, 2.0]
NSAMPLES = [16, 16, 32]
EPS = 1e-5


def _fps(xyz, npoint):
    b, n, _ = xyz.shape

    def body(i, state):
        dists, farthest, idxs = state
        idxs = idxs.at[:, i].set(farthest)
        centroid = jnp.take_along_axis(xyz, farthest[:, None, None], axis=1)
        d = jnp.sum((xyz - centroid) ** 2, axis=-1)
        dists = jnp.minimum(dists, d)
        farthest = jnp.argmax(dists, axis=-1).astype(jnp.int32)
        return (dists, farthest, idxs)

    state = (jnp.full((b, n), 1e10, jnp.float32), jnp.zeros((b,), jnp.int32),
             jnp.zeros((b, npoint), jnp.int32))
    state = jax.lax.fori_loop(0, npoint, body, state)
    return state[2]


def _ball_query(radius, nsample, xyz, new_xyz):
    n = xyz.shape[1]
    d2 = jnp.sum((new_xyz[:, :, None, :] - xyz[:, None, :, :]) ** 2, axis=-1)
    mask = d2 <= radius * radius
    key_arr = jnp.where(mask, jnp.arange(n, dtype=jnp.int32)[None, None, :], n)
    idx_sorted = jnp.sort(key_arr, axis=-1)[..., :nsample]
    first = idx_sorted[..., :1]
    idx = jnp.where(idx_sorted >= n, jnp.broadcast_to(first, idx_sorted.shape), idx_sorted)
    idx = jnp.where(idx >= n, 0, idx)
    return idx.astype(jnp.int32)


def _group(radius, nsample, xyz, new_xyz, features):
    idx = _ball_query(radius, nsample, xyz, new_xyz)
    grouped_xyz = jax.vmap(lambda pts, ix: pts[ix])(xyz, idx)
    grouped_xyz = grouped_xyz - new_xyz[:, :, None, :]
    grouped_xyz = jnp.transpose(grouped_xyz, (0, 3, 1, 2))
    grouped_feat = jax.vmap(lambda f, ix: f[:, ix])(features, idx)
    return jnp.concatenate([grouped_xyz, grouped_feat], axis=1)


def _mlp_branch(x, layers):
    for L in layers:
        x = jnp.einsum('oc,bcms->boms', L["W"], x)
        mean = jnp.mean(x, axis=(0, 2, 3), keepdims=True)
        var = jnp.var(x, axis=(0, 2, 3), keepdims=True)
        x = (x - mean) / jnp.sqrt(var + EPS)
        x = x * L["g"][None, :, None, None] + L["b"][None, :, None, None]
        x = jax.nn.relu(x)
    return x


def _out_kernel(x_ref, w_ref, g_ref, b_ref, o_ref):
    ys = []
    for b in range(B):
        ys.append(jnp.dot(w_ref[...], x_ref[b], preferred_element_type=jnp.float32))
    y = jnp.stack(ys, axis=0)
    cnt = y.shape[0] * y.shape[2]
    mean = jnp.sum(y, axis=(0, 2)) / cnt
    var = jnp.sum(y * y, axis=(0, 2)) / cnt - mean * mean
    scale = g_ref[...] / jnp.sqrt(var + EPS)
    shift = b_ref[...] - mean * scale
    o_ref[...] = jnp.maximum(y * scale[None, :, None] + shift[None, :, None], 0.0)


def _out_layer(nf, L):
    O, Cin = L["W"].shape
    M = nf.shape[2]
    return pl.pallas_call(
        _out_kernel,
        out_shape=jax.ShapeDtypeStruct((B, O, M), jnp.float32),
    )(nf, L["W"], L["g"], L["b"])


def kernel(xyz, features, params):
    fps_idx = _fps(xyz, NPOINT)
    new_xyz = jax.vmap(lambda pts, ix: pts[ix])(xyz, fps_idx)
    feats = []
    for i in range(len(RADII)):
        g = _group(RADII[i], NSAMPLES[i], xyz, new_xyz, features)
        h = _mlp_branch(g, params["branches"][i])
        h = jnp.max(h, axis=-1)
        feats.append(h)
    nf = jnp.concatenate(feats, axis=1)
    nf = _out_layer(nf, params["out"])
    return (new_xyz, nf)
